# Initial kernel scaffold; baseline (speedup 1.0000x reference)
import jax
import jax.numpy as jnp
from jax.experimental import pallas as pl


def kernel(x, table):
    raise NotImplementedError("write your pallas kernel here")


# speedup vs baseline: 3.3444x; 3.3444x over previous
import functools

import jax
import jax.numpy as jnp
from jax import lax
from jax.experimental import pallas as pl
from jax.experimental.pallas import tpu as pltpu
from jax.experimental.pallas import tpu_sc as plsc

_NC = 2
_NS = 16
_NW = _NC * _NS

_D = 128
_CHUNK = 128
_NBUF = 2


@functools.lru_cache(maxsize=None)
def _build(n_total, d):
    per_w = n_total // _NW
    n_chunks = per_w // _CHUNK
    assert per_w * _NW == n_total and n_chunks * _CHUNK == per_w
    assert n_chunks % _NBUF == 0

    mesh = plsc.VectorSubcoreMesh(core_axis_name="c", subcore_axis_name="s")

    @functools.partial(
        pl.kernel,
        out_type=jax.ShapeDtypeStruct((n_total, d), jnp.float32),
        mesh=mesh,
        scratch_types=[
            pltpu.VMEM((n_chunks, _CHUNK), jnp.int32),
            pltpu.VMEM((_NBUF, _CHUNK, d), jnp.float32),
            pltpu.SemaphoreType.DMA,
            pltpu.SemaphoreType.DMA,
            pltpu.SemaphoreType.DMA,
            pltpu.SemaphoreType.DMA,
        ],
    )
    def gather_kernel(idx_hbm, table_hbm, out_hbm, idx_v, rows_v, g0, g1, o0, o1):
        wid = lax.axis_index("s") * _NC + lax.axis_index("c")
        base = wid * per_w
        gsem = (g0, g1)
        osem = (o0, o1)

        pltpu.sync_copy(idx_hbm.at[wid], idx_v)

        def start_gather(i, b):
            pltpu.async_copy(table_hbm.at[idx_v.at[i]], rows_v.at[b], gsem[b])

        def wait_gather(b):
            pltpu.make_async_copy(
                table_hbm.at[pl.ds(0, _CHUNK)], rows_v.at[b], gsem[b]
            ).wait()

        def start_out(i, b):
            pltpu.async_copy(
                rows_v.at[b], out_hbm.at[pl.ds(base + i * _CHUNK, _CHUNK)], osem[b]
            )

        def wait_out(b):
            pltpu.make_async_copy(
                rows_v.at[b], out_hbm.at[pl.ds(0, _CHUNK)], osem[b]
            ).wait()

        for b in range(_NBUF):
            start_gather(b, b)

        @pl.loop(0, n_chunks - _NBUF, step=_NBUF)
        def _steady(g):
            for b in range(_NBUF):
                i = g + b
                wait_gather(b)
                start_out(i, b)
                wait_out(b)
                start_gather(i + _NBUF, b)

        for b in range(_NBUF):
            i = n_chunks - _NBUF + b
            wait_gather(b)
            start_out(i, b)
        for b in range(_NBUF):
            wait_out(b)

    return gather_kernel, per_w, n_chunks


def kernel(x, table):
    batch, hist = x.shape
    n_total = batch * hist
    d = table.shape[1]
    gather_kernel, per_w, n_chunks = _build(n_total, d)
    idx = x.reshape(_NW, n_chunks, _CHUNK).astype(jnp.int32)
    out = gather_kernel(idx, table)
    return out.reshape(batch, hist, d)


# speedup vs baseline: 3.3569x; 1.0037x over previous
import functools

import jax
import jax.numpy as jnp
from jax import lax
from jax.experimental import pallas as pl
from jax.experimental.pallas import tpu as pltpu
from jax.experimental.pallas import tpu_sc as plsc

_NC = 2
_NS = 16
_NW = _NC * _NS

_D = 128
_CHUNK = 128
_NBUF = 5


@functools.lru_cache(maxsize=None)
def _build(n_total, d):
    per_w = n_total // _NW
    n_chunks = per_w // _CHUNK
    assert per_w * _NW == n_total and n_chunks * _CHUNK == per_w
    assert n_chunks % _NBUF == 0

    mesh = plsc.VectorSubcoreMesh(core_axis_name="c", subcore_axis_name="s")

    @functools.partial(
        pl.kernel,
        out_type=jax.ShapeDtypeStruct((n_total, d), jnp.float32),
        mesh=mesh,
        scratch_types=[
            pltpu.VMEM((n_chunks, _CHUNK), jnp.int32),
            pltpu.VMEM((_NBUF, _CHUNK, d), jnp.float32),
        ]
        + [pltpu.SemaphoreType.DMA] * (2 * _NBUF),
    )
    def gather_kernel(idx_hbm, table_hbm, out_hbm, idx_v, rows_v, *sems):
        wid = lax.axis_index("s") * _NC + lax.axis_index("c")
        base = wid * per_w
        gsem = sems[:_NBUF]
        osem = sems[_NBUF:]

        pltpu.sync_copy(idx_hbm.at[wid], idx_v)

        def start_gather(i, b):
            pltpu.async_copy(table_hbm.at[idx_v.at[i]], rows_v.at[b], gsem[b])

        def wait_gather(b):
            pltpu.make_async_copy(
                table_hbm.at[pl.ds(0, _CHUNK)], rows_v.at[b], gsem[b]
            ).wait()

        def start_out(i, b):
            pltpu.async_copy(
                rows_v.at[b], out_hbm.at[pl.ds(base + i * _CHUNK, _CHUNK)], osem[b]
            )

        def wait_out(b):
            pltpu.make_async_copy(
                rows_v.at[b], out_hbm.at[pl.ds(0, _CHUNK)], osem[b]
            ).wait()

        for b in range(_NBUF):
            start_gather(b, b)

        @pl.loop(0, n_chunks - _NBUF, step=_NBUF)
        def _steady(g):
            for b in range(_NBUF):
                i = g + b
                wait_gather(b)
                start_out(i, b)
                wait_out(b)
                start_gather(i + _NBUF, b)

        for b in range(_NBUF):
            i = n_chunks - _NBUF + b
            wait_gather(b)
            start_out(i, b)
        for b in range(_NBUF):
            wait_out(b)

    return gather_kernel, per_w, n_chunks


def kernel(x, table):
    batch, hist = x.shape
    n_total = batch * hist
    d = table.shape[1]
    gather_kernel, per_w, n_chunks = _build(n_total, d)
    idx = x.reshape(_NW, n_chunks, _CHUNK).astype(jnp.int32)
    out = gather_kernel(idx, table)
    return out.reshape(batch, hist, d)


# speedup vs baseline: 5.9982x; 1.7868x over previous
import functools

import jax
import jax.numpy as jnp
from jax import lax
from jax.experimental import pallas as pl
from jax.experimental.pallas import tpu as pltpu
from jax.experimental.pallas import tpu_sc as plsc

_NC = 2
_NS = 16
_NW = _NC * _NS

_PAD = 64
_NBUF = 8


@functools.lru_cache(maxsize=None)
def _build(batch, hist, d):
    per_w = batch // _NW
    assert per_w * _NW == batch and per_w % _NBUF == 0 and hist <= 128

    mesh = plsc.VectorSubcoreMesh(core_axis_name="c", subcore_axis_name="s")

    @functools.partial(
        pl.kernel,
        out_type=jax.ShapeDtypeStruct((batch, hist, d), jnp.float32),
        mesh=mesh,
        scratch_types=[
            pltpu.VMEM((per_w * _PAD,), jnp.int32),
            pltpu.VMEM((_NBUF, hist, d), jnp.float32),
        ]
        + [pltpu.SemaphoreType.DMA] * (2 * _NBUF),
    )
    def gather_kernel(idx_hbm, table_hbm, out_hbm, idx_v, rows_v, *sems):
        wid = lax.axis_index("s") * _NC + lax.axis_index("c")
        base_b = wid * per_w
        gsem = sems[:_NBUF]
        osem = sems[_NBUF:]

        pltpu.sync_copy(idx_hbm.at[pl.ds(wid * per_w * _PAD, per_w * _PAD)], idx_v)

        def start_gather(i, b):
            pltpu.async_copy(
                table_hbm.at[idx_v.at[pl.ds(i * _PAD, hist)]], rows_v.at[b], gsem[b]
            )

        def wait_gather(b):
            pltpu.make_async_copy(out_hbm.at[0], rows_v.at[b], gsem[b]).wait()

        def start_out(i, b):
            pltpu.async_copy(rows_v.at[b], out_hbm.at[base_b + i], osem[b])

        def wait_out(b):
            pltpu.make_async_copy(rows_v.at[b], out_hbm.at[0], osem[b]).wait()

        for b in range(_NBUF):
            start_gather(b, b)

        @pl.loop(0, per_w - _NBUF, step=_NBUF)
        def _steady(g):
            for b in range(_NBUF):
                i = g + b
                wait_gather(b)
                start_out(i, b)
                wait_out(b)
                start_gather(i + _NBUF, b)

        for b in range(_NBUF):
            i = per_w - _NBUF + b
            wait_gather(b)
            start_out(i, b)
        for b in range(_NBUF):
            wait_out(b)

    return gather_kernel, per_w


def kernel(x, table):
    batch, hist = x.shape
    d = table.shape[1]
    gather_kernel, per_w = _build(batch, hist, d)
    idx = jnp.pad(x.astype(jnp.int32), ((0, 0), (0, _PAD - hist)))
    idx = idx.reshape(batch * _PAD)
    return gather_kernel(idx, table)


# speedup vs baseline: 10.4377x; 1.7401x over previous
import functools

import jax
import jax.numpy as jnp
from jax import lax
from jax.experimental import pallas as pl
from jax.experimental.pallas import tpu as pltpu
from jax.experimental.pallas import tpu_sc as plsc

_NC = 2
_NS = 16
_NW = _NC * _NS

_CHUNK = 128
_NBUF = 5


@functools.lru_cache(maxsize=None)
def _build(batch, hist, d):
    n_total = batch * hist
    per_w = n_total // _NW
    n_chunks = per_w // _CHUNK
    wph = batch // _CHUNK
    assert per_w * _NW == n_total and n_chunks * _CHUNK == per_w
    assert wph * _CHUNK == batch and n_chunks % _NBUF == 0

    mesh = plsc.VectorSubcoreMesh(core_axis_name="c", subcore_axis_name="s")

    @functools.partial(
        pl.kernel,
        out_type=jax.ShapeDtypeStruct((hist, batch, d), jnp.float32),
        mesh=mesh,
        scratch_types=[
            pltpu.VMEM((per_w,), jnp.int32),
            pltpu.VMEM((_NBUF, _CHUNK, d), jnp.float32),
        ]
        + [pltpu.SemaphoreType.DMA] * (2 * _NBUF),
    )
    def gather_kernel(idx_hbm, table_hbm, out_hbm, idx_v, rows_v, *sems):
        wid = lax.axis_index("s") * _NC + lax.axis_index("c")
        gsem = sems[:_NBUF]
        osem = sems[_NBUF:]

        pltpu.sync_copy(idx_hbm.at[pl.ds(wid * per_w, per_w)], idx_v)

        k0 = wid * n_chunks

        def start_gather(i, b):
            pltpu.async_copy(
                table_hbm.at[idx_v.at[pl.ds(i * _CHUNK, _CHUNK)]],
                rows_v.at[b],
                gsem[b],
            )

        def wait_gather(b):
            pltpu.make_async_copy(
                table_hbm.at[pl.ds(0, _CHUNK)], rows_v.at[b], gsem[b]
            ).wait()

        def start_out(i, b):
            k = k0 + i
            h = k // wph
            b0 = (k % wph) * _CHUNK
            pltpu.async_copy(
                rows_v.at[b], out_hbm.at[h, pl.ds(b0, _CHUNK)], osem[b]
            )

        def wait_out(b):
            pltpu.make_async_copy(
                rows_v.at[b], out_hbm.at[0, pl.ds(0, _CHUNK)], osem[b]
            ).wait()

        for b in range(_NBUF):
            start_gather(b, b)

        @pl.loop(0, n_chunks - _NBUF, step=_NBUF)
        def _steady(g):
            for b in range(_NBUF):
                i = g + b
                wait_gather(b)
                start_out(i, b)
                wait_out(b)
                start_gather(i + _NBUF, b)

        for b in range(_NBUF):
            i = n_chunks - _NBUF + b
            wait_gather(b)
            start_out(i, b)
        for b in range(_NBUF):
            wait_out(b)

    return gather_kernel


def kernel(x, table):
    batch, hist = x.shape
    d = table.shape[1]
    gather_kernel = _build(batch, hist, d)
    idx = x.T.reshape(batch * hist).astype(jnp.int32)
    out_t = gather_kernel(idx, table)
    return jnp.transpose(out_t, (1, 0, 2))


# speedup vs baseline: 10.4651x; 1.0026x over previous
import functools

import jax
import jax.numpy as jnp
from jax import lax
from jax.experimental import pallas as pl
from jax.experimental.pallas import tpu as pltpu
from jax.experimental.pallas import tpu_sc as plsc

_NC = 2
_NS = 16
_NW = _NC * _NS

_CHUNK = 128
_NBUF = 5


@functools.lru_cache(maxsize=None)
def _build(batch, hist, d):
    n_total = batch * hist
    per_w = n_total // _NW
    n_chunks = per_w // _CHUNK
    wph = batch // _CHUNK
    per_w_pad = 8192
    assert per_w * _NW == n_total and n_chunks * _CHUNK == per_w
    assert wph * _CHUNK == batch and n_chunks % _NBUF == 0 and per_w <= per_w_pad

    mesh = plsc.VectorSubcoreMesh(core_axis_name="c", subcore_axis_name="s")

    @functools.partial(
        pl.kernel,
        out_type=jax.ShapeDtypeStruct((hist, batch, d), jnp.float32),
        mesh=mesh,
        scratch_types=[
            pltpu.VMEM((per_w,), jnp.int32),
            pltpu.VMEM((_NBUF, _CHUNK, d), jnp.float32),
        ]
        + [pltpu.SemaphoreType.DMA] * (2 * _NBUF),
    )
    def gather_kernel(idx_hbm, table_hbm, out_hbm, idx_v, rows_v, *sems):
        wid = lax.axis_index("s") * _NC + lax.axis_index("c")
        gsem = sems[:_NBUF]
        osem = sems[_NBUF:]

        pltpu.sync_copy(idx_hbm.at[pl.ds(wid * per_w_pad, per_w)], idx_v)

        k0 = wid * n_chunks

        def start_gather(i, b):
            pltpu.async_copy(
                table_hbm.at[idx_v.at[pl.ds(i * _CHUNK, _CHUNK)]],
                rows_v.at[b],
                gsem[b],
            )

        def wait_gather(b):
            pltpu.make_async_copy(
                table_hbm.at[pl.ds(0, _CHUNK)], rows_v.at[b], gsem[b]
            ).wait()

        def start_out(i, b):
            k = k0 + i
            h = k // wph
            b0 = (k % wph) * _CHUNK
            pltpu.async_copy(
                rows_v.at[b], out_hbm.at[h, pl.ds(b0, _CHUNK)], osem[b]
            )

        def wait_out(b):
            pltpu.make_async_copy(
                rows_v.at[b], out_hbm.at[0, pl.ds(0, _CHUNK)], osem[b]
            ).wait()

        for b in range(_NBUF):
            start_gather(b, b)

        @pl.loop(0, n_chunks - _NBUF, step=_NBUF)
        def _steady(g):
            for b in range(_NBUF):
                i = g + b
                wait_gather(b)
                start_out(i, b)
                wait_out(b)
                start_gather(i + _NBUF, b)

        for b in range(_NBUF):
            i = n_chunks - _NBUF + b
            wait_gather(b)
            start_out(i, b)
        for b in range(_NBUF):
            wait_out(b)

    return gather_kernel


def kernel(x, table):
    batch, hist = x.shape
    d = table.shape[1]
    gather_kernel = _build(batch, hist, d)
    per_w = batch * hist // _NW
    idx = x.T.reshape(_NW, per_w).astype(jnp.int32)
    idx = jnp.pad(idx, ((0, 0), (0, 8192 - per_w))).reshape(_NW * 8192)
    out_t = gather_kernel(idx, table)
    return jnp.transpose(out_t, (1, 0, 2))


# speedup vs baseline: 10.7438x; 1.0266x over previous
import functools

import jax
import jax.numpy as jnp
from jax import lax
from jax.experimental import pallas as pl
from jax.experimental.pallas import tpu as pltpu
from jax.experimental.pallas import tpu_sc as plsc

_NC = 2
_NS = 16
_NW = _NC * _NS

_NBUF = 5


@functools.lru_cache(maxsize=None)
def _build(batch, hist, d):
    blk = batch // _NW
    assert blk * _NW == batch and blk <= 128 and blk % 8 == 0
    assert hist % _NBUF == 0

    mesh = plsc.VectorSubcoreMesh(core_axis_name="c", subcore_axis_name="s")

    @functools.partial(
        pl.kernel,
        out_type=jax.ShapeDtypeStruct((hist, batch, d), jnp.float32),
        mesh=mesh,
        scratch_types=[
            pltpu.VMEM((hist, blk), jnp.int32),
            pltpu.VMEM((_NBUF, blk, d), jnp.float32),
        ]
        + [pltpu.SemaphoreType.DMA] * (2 * _NBUF),
    )
    def gather_kernel(idx_hbm, table_hbm, out_hbm, idx_v, rows_v, *sems):
        wid = lax.axis_index("s") * _NC + lax.axis_index("c")
        b0 = wid * blk
        gsem = sems[:_NBUF]
        osem = sems[_NBUF:]

        pltpu.sync_copy(idx_hbm.at[:, pl.ds(b0, blk)], idx_v)

        def start_gather(h, b):
            pltpu.async_copy(
                table_hbm.at[idx_v.at[h]], rows_v.at[b], gsem[b]
            )

        def wait_gather(b):
            pltpu.make_async_copy(
                table_hbm.at[pl.ds(0, blk)], rows_v.at[b], gsem[b]
            ).wait()

        def start_out(h, b):
            pltpu.async_copy(
                rows_v.at[b], out_hbm.at[h, pl.ds(b0, blk)], osem[b]
            )

        def wait_out(b):
            pltpu.make_async_copy(
                rows_v.at[b], out_hbm.at[0, pl.ds(0, blk)], osem[b]
            ).wait()

        for b in range(_NBUF):
            start_gather(b, b)

        @pl.loop(0, hist - _NBUF, step=_NBUF)
        def _steady(g):
            for b in range(_NBUF):
                h = g + b
                wait_gather(b)
                start_out(h, b)
                wait_out(b)
                start_gather(h + _NBUF, b)

        for b in range(_NBUF):
            h = hist - _NBUF + b
            wait_gather(b)
            start_out(h, b)
        for b in range(_NBUF):
            wait_out(b)

    return gather_kernel


def kernel(x, table):
    batch, hist = x.shape
    d = table.shape[1]
    gather_kernel = _build(batch, hist, d)
    out_t = gather_kernel(x.T.astype(jnp.int32), table)
    return jnp.transpose(out_t, (1, 0, 2))


# speedup vs baseline: 10.8185x; 1.0070x over previous
import functools

import jax
import jax.numpy as jnp
from jax import lax
from jax.experimental import pallas as pl
from jax.experimental.pallas import tpu as pltpu
from jax.experimental.pallas import tpu_sc as plsc

_NC = 2
_NS = 16
_NW = _NC * _NS

_NBUF = 4
_NSP = 2


@functools.lru_cache(maxsize=None)
def _build(batch, hist, d):
    blk = batch // _NW
    assert blk * _NW == batch and blk <= 128 and blk % 8 == 0
    n_steady = max(0, (hist + 1 - 2 * _NBUF) // _NBUF * _NBUF)
    drain_lo = _NBUF + n_steady
    assert hist >= _NBUF + 2

    mesh = plsc.VectorSubcoreMesh(core_axis_name="c", subcore_axis_name="s")

    @functools.partial(
        pl.kernel,
        out_type=jax.ShapeDtypeStruct((hist, batch, d), jnp.float32),
        mesh=mesh,
        scratch_types=[
            pltpu.VMEM((hist, blk), jnp.int32),
            pltpu.VMEM((_NBUF, blk, d), jnp.float32),
            pltpu.VMEM_SHARED((_NS, _NSP, blk, d), jnp.float32),
        ]
        + [pltpu.SemaphoreType.DMA] * (_NBUF + 2 * _NSP),
    )
    def gather_kernel(idx_hbm, table_hbm, out_hbm, idx_v, rows_v, sp_v, *sems):
        wid = lax.axis_index("s") * _NC + lax.axis_index("c")
        sid = lax.axis_index("s")
        b0 = wid * blk
        gsem = sems[:_NBUF]
        csem = sems[_NBUF : _NBUF + _NSP]
        osem = sems[_NBUF + _NSP :]

        pltpu.sync_copy(idx_hbm.at[:, pl.ds(b0, blk)], idx_v)

        def start_gather(h, b):
            pltpu.async_copy(table_hbm.at[idx_v.at[h]], rows_v.at[b], gsem[b])

        def wait_gather(b):
            pltpu.make_async_copy(
                table_hbm.at[pl.ds(0, blk)], rows_v.at[b], gsem[b]
            ).wait()

        def start_copy(b, s):
            pltpu.async_copy(rows_v.at[b], sp_v.at[sid, s], csem[s])

        def wait_copy(s):
            pltpu.make_async_copy(rows_v.at[0], sp_v.at[sid, s], csem[s]).wait()

        def start_out(h, s):
            pltpu.async_copy(
                sp_v.at[sid, s], out_hbm.at[h, pl.ds(b0, blk)], osem[s]
            )

        def wait_out(s):
            pltpu.make_async_copy(
                sp_v.at[sid, s], out_hbm.at[0, pl.ds(0, blk)], osem[s]
            ).wait()

        for b in range(_NBUF):
            start_gather(b, b)

        def stage_a(i, first=False):
            wait_gather(i % _NBUF)
            if not first:
                wait_out(i % _NSP)
            start_copy(i % _NBUF, i % _NSP)

        def stage_b(i, refill=True):
            wait_copy(i % _NSP)
            start_out(i, i % _NSP)
            if refill:
                start_gather(i + _NBUF, i % _NBUF)

        stage_a(0, first=True)
        stage_a(1, first=_NSP > 1)
        stage_b(0)
        for i in range(2, _NBUF):
            stage_a(i)
            stage_b(i - 1)

        @pl.loop(_NBUF, _NBUF + n_steady, step=_NBUF)
        def _steady(g):
            for j in range(_NBUF):
                i = g + j
                wait_gather(j)
                wait_out(j % _NSP)
                start_copy(j, j % _NSP)
                jp = (j - 1) % _NBUF
                wait_copy((j - 1) % _NSP)
                start_out(i - 1, (j - 1) % _NSP)
                start_gather(i - 1 + _NBUF, jp)

        for i in range(drain_lo, hist):
            stage_a(i)
            stage_b(i - 1, refill=(i - 1 + _NBUF < hist))
        stage_b(hist - 1, refill=False)
        for s in range(_NSP):
            wait_out(s)

    return gather_kernel


def kernel(x, table):
    batch, hist = x.shape
    d = table.shape[1]
    gather_kernel = _build(batch, hist, d)
    out_t = gather_kernel(x.T.astype(jnp.int32), table)
    return jnp.transpose(out_t, (1, 0, 2))
